# Initial kernel scaffold; baseline (speedup 1.0000x reference)
import jax
import jax.numpy as jnp
from jax.experimental import pallas as pl


def kernel(x, w_gate, W_exp, b_exp):
    raise NotImplementedError("write your pallas kernel here")


# speedup vs baseline: 2.0301x; 2.0301x over previous
import jax
import jax.numpy as jnp
from jax.experimental import pallas as pl
from jax.experimental.pallas import tpu as pltpu


def _gating_body(x_ref, wg_ref, gates_ref, loss_ref, gx_ref):
    i = pl.program_id(0)
    n = pl.num_programs(0)
    gb = x_ref.shape[0]
    base = pl.multiple_of(i * gb, 8)
    gx_ref[pl.ds(base, gb), :] = jnp.mean(x_ref[...], axis=2)

    @pl.when(i == n - 1)
    def _():
        B = gx_ref.shape[0]
        E = wg_ref.shape[1]
        logits = jnp.dot(gx_ref[...], wg_ref[...],
                         preferred_element_type=jnp.float32)
        z = jnp.exp(logits - jnp.max(logits, axis=1, keepdims=True))
        sm = z / jnp.sum(z, axis=1, keepdims=True)
        iota = jax.lax.broadcasted_iota(jnp.int32, (B, E), 1)
        m1 = jnp.max(sm, axis=1, keepdims=True)
        i1 = jnp.min(jnp.where(sm == m1, iota, E), axis=1, keepdims=True)
        masked = jnp.where(iota == i1, -1.0, sm)
        m2 = jnp.max(masked, axis=1, keepdims=True)
        i2 = jnp.min(jnp.where(masked == m2, iota, E), axis=1, keepdims=True)
        denom = m1 + m2 + 1e-6
        gates = (jnp.where(iota == i1, m1 / denom, 0.0)
                 + jnp.where(iota == i2, m2 / denom, 0.0))
        gates_ref[...] = gates

        def cv_sq(v):
            mu = jnp.mean(v, keepdims=True)
            var = jnp.sum((v - mu) ** 2, keepdims=True) / (E - 1)
            return var / (mu ** 2 + 1e-10)

        imp = jnp.sum(gates, axis=0, keepdims=True)
        load = jnp.sum((gates > 0.0).astype(jnp.float32), axis=0,
                       keepdims=True)
        loss_ref[...] = (cv_sq(imp) + cv_sq(load)) * 0.01


def _combine_body(g_ref, w2_ref, be_ref, wc_ref, bc_ref):
    j = pl.program_id(0)
    wc_ref[...] = jnp.dot(g_ref[...], w2_ref[...],
                          preferred_element_type=jnp.float32)

    @pl.when(j == 0)
    def _():
        bc_ref[...] = jnp.dot(g_ref[...], be_ref[...],
                              preferred_element_type=jnp.float32)


def _make_conv_body(C_IN, C_OUT, H, W, KS):
    Wp = W + 2
    FLAT = (H + 2) * Wp
    PADL = Wp + 1

    def conv_body(x_ref, wc_ref, bc_ref, out_ref, xb_ref, acc_ref):
        b = pl.program_id(0)

        @pl.when(b == 0)
        def _():
            xb_ref[...] = jnp.zeros_like(xb_ref)

        for h in range(H):
            off = PADL + Wp * (h + 1) + 1
            xb_ref[:, off:off + W] = x_ref[0, :, h, :]

        wc = wc_ref[0]
        for t in range(KS * KS):
            kh, kw = t // KS, t % KS
            d = PADL + Wp * (kh - 1) + (kw - 1)
            xs = xb_ref[:, d:d + FLAT]
            wt = wc[t * C_OUT:(t + 1) * C_OUT, :]
            p = jnp.dot(wt, xs, preferred_element_type=jnp.float32)
            if t == 0:
                acc_ref[...] = p
            else:
                acc_ref[...] += p

        bias = bc_ref[0]
        for h in range(H):
            off = Wp * (h + 1) + 1
            out_ref[0, :, h, :] = acc_ref[:, off:off + W] + bias

    return conv_body


def kernel(x, w_gate, W_exp, b_exp):
    B, C_IN, H, W = x.shape
    E = w_gate.shape[1]
    C_OUT = W_exp.shape[1]
    KS = W_exp.shape[3]
    HW = H * W
    f32 = jnp.float32

    GB = 8
    x3 = x.reshape(B, C_IN, HW)
    gates, loss_arr = pl.pallas_call(
        _gating_body,
        grid=(B // GB,),
        in_specs=[
            pl.BlockSpec((GB, C_IN, HW), lambda i: (i, 0, 0)),
            pl.BlockSpec((C_IN, E), lambda i: (0, 0)),
        ],
        out_specs=[
            pl.BlockSpec((B, E), lambda i: (0, 0)),
            pl.BlockSpec((1, 1), lambda i: (0, 0)),
        ],
        out_shape=[
            jax.ShapeDtypeStruct((B, E), f32),
            jax.ShapeDtypeStruct((1, 1), f32),
        ],
        scratch_shapes=[pltpu.VMEM((B, C_IN), f32)],
    )(x3, w_gate)

    W2 = W_exp.transpose(0, 3, 4, 1, 2).reshape(E, KS * KS * C_OUT * C_IN)
    NCH = 8
    CHUNK = W2.shape[1] // NCH
    W_c, b_c = pl.pallas_call(
        _combine_body,
        grid=(NCH,),
        in_specs=[
            pl.BlockSpec((B, E), lambda j: (0, 0)),
            pl.BlockSpec((E, CHUNK), lambda j: (0, j)),
            pl.BlockSpec((E, C_OUT), lambda j: (0, 0)),
        ],
        out_specs=[
            pl.BlockSpec((B, CHUNK), lambda j: (0, j)),
            pl.BlockSpec((B, C_OUT), lambda j: (0, 0)),
        ],
        out_shape=[
            jax.ShapeDtypeStruct((B, KS * KS * C_OUT * C_IN), f32),
            jax.ShapeDtypeStruct((B, C_OUT), f32),
        ],
    )(gates, W2, b_exp)

    Wc3 = W_c.reshape(B, KS * KS * C_OUT, C_IN)
    bc3 = b_c.reshape(B, C_OUT, 1)
    FLAT = (H + 2) * (W + 2)
    PADL = (W + 2) + 1
    y = pl.pallas_call(
        _make_conv_body(C_IN, C_OUT, H, W, KS),
        grid=(B,),
        in_specs=[
            pl.BlockSpec((1, C_IN, H, W), lambda b: (b, 0, 0, 0)),
            pl.BlockSpec((1, KS * KS * C_OUT, C_IN), lambda b: (b, 0, 0)),
            pl.BlockSpec((1, C_OUT, 1), lambda b: (b, 0, 0)),
        ],
        out_specs=pl.BlockSpec((1, C_OUT, H, W), lambda b: (b, 0, 0, 0)),
        out_shape=jax.ShapeDtypeStruct((B, C_OUT, H, W), f32),
        scratch_shapes=[
            pltpu.VMEM((C_IN, FLAT + 2 * PADL), f32),
            pltpu.VMEM((C_OUT, FLAT), f32),
        ],
    )(x, Wc3, bc3)

    return (y, loss_arr[0, 0])


# speedup vs baseline: 3.1488x; 1.5511x over previous
import jax
import jax.numpy as jnp
from jax.experimental import pallas as pl
from jax.experimental.pallas import tpu as pltpu


def _gating_body(x_ref, wg_ref, gates_ref, loss_ref, gx_ref):
    i = pl.program_id(0)
    n = pl.num_programs(0)
    gb = x_ref.shape[0]
    base = pl.multiple_of(i * gb, 8)
    gx_ref[pl.ds(base, gb), :] = jnp.mean(x_ref[...], axis=2)

    @pl.when(i == n - 1)
    def _():
        B = gx_ref.shape[0]
        E = wg_ref.shape[1]
        logits = jnp.dot(gx_ref[...], wg_ref[...],
                         preferred_element_type=jnp.float32)
        z = jnp.exp(logits - jnp.max(logits, axis=1, keepdims=True))
        sm = z / jnp.sum(z, axis=1, keepdims=True)
        iota = jax.lax.broadcasted_iota(jnp.int32, (B, E), 1)
        m1 = jnp.max(sm, axis=1, keepdims=True)
        i1 = jnp.min(jnp.where(sm == m1, iota, E), axis=1, keepdims=True)
        masked = jnp.where(iota == i1, -1.0, sm)
        m2 = jnp.max(masked, axis=1, keepdims=True)
        i2 = jnp.min(jnp.where(masked == m2, iota, E), axis=1, keepdims=True)
        denom = m1 + m2 + 1e-6
        gates = (jnp.where(iota == i1, m1 / denom, 0.0)
                 + jnp.where(iota == i2, m2 / denom, 0.0))
        gates_ref[...] = gates

        def cv_sq(v):
            mu = jnp.mean(v, keepdims=True)
            var = jnp.sum((v - mu) ** 2, keepdims=True) / (E - 1)
            return var / (mu ** 2 + 1e-10)

        imp = jnp.sum(gates, axis=0, keepdims=True)
        load = jnp.sum((gates > 0.0).astype(jnp.float32), axis=0,
                       keepdims=True)
        loss_ref[...] = (cv_sq(imp) + cv_sq(load)) * 0.01


def _combine_body(g_ref, w2_ref, be_ref, wc_ref, bc_ref):
    j = pl.program_id(0)
    wc_ref[...] = jnp.dot(g_ref[...], w2_ref[...],
                          preferred_element_type=jnp.float32)

    @pl.when(j == 0)
    def _():
        bc_ref[...] = jnp.dot(g_ref[...], be_ref[...],
                              preferred_element_type=jnp.float32)


def _make_conv_body(C_IN, C_OUT, H, W, KS):
    Wp = W + 2
    FLAT = (H + 2) * Wp
    PADL = Wp + 1

    def conv_body(x_ref, wc_ref, bc_ref, out_ref):
        wc = wc_ref[0]
        for t in range(KS * KS):
            kh, kw = t // KS, t % KS
            d = PADL + Wp * (kh - 1) + (kw - 1)
            xs = x_ref[0, :, d:d + FLAT]
            wt = wc[t * C_OUT:(t + 1) * C_OUT, :]
            p = jnp.dot(wt, xs, preferred_element_type=jnp.float32)
            if t == 0:
                out_ref[0] = p
            else:
                out_ref[0] += p
        out_ref[0] += bc_ref[0]

    return conv_body


def kernel(x, w_gate, W_exp, b_exp):
    B, C_IN, H, W = x.shape
    E = w_gate.shape[1]
    C_OUT = W_exp.shape[1]
    KS = W_exp.shape[3]
    HW = H * W
    f32 = jnp.float32

    GB = 8
    x3 = x.reshape(B, C_IN, HW)
    gates, loss_arr = pl.pallas_call(
        _gating_body,
        grid=(B // GB,),
        in_specs=[
            pl.BlockSpec((GB, C_IN, HW), lambda i: (i, 0, 0)),
            pl.BlockSpec((C_IN, E), lambda i: (0, 0)),
        ],
        out_specs=[
            pl.BlockSpec((B, E), lambda i: (0, 0)),
            pl.BlockSpec((1, 1), lambda i: (0, 0)),
        ],
        out_shape=[
            jax.ShapeDtypeStruct((B, E), f32),
            jax.ShapeDtypeStruct((1, 1), f32),
        ],
        scratch_shapes=[pltpu.VMEM((B, C_IN), f32)],
    )(x3, w_gate)

    W2 = W_exp.transpose(0, 3, 4, 1, 2).reshape(E, KS * KS * C_OUT * C_IN)
    NCH = 8
    CHUNK = W2.shape[1] // NCH
    W_c, b_c = pl.pallas_call(
        _combine_body,
        grid=(NCH,),
        in_specs=[
            pl.BlockSpec((B, E), lambda j: (0, 0)),
            pl.BlockSpec((E, CHUNK), lambda j: (0, j)),
            pl.BlockSpec((E, C_OUT), lambda j: (0, 0)),
        ],
        out_specs=[
            pl.BlockSpec((B, CHUNK), lambda j: (0, j)),
            pl.BlockSpec((B, C_OUT), lambda j: (0, 0)),
        ],
        out_shape=[
            jax.ShapeDtypeStruct((B, KS * KS * C_OUT * C_IN), f32),
            jax.ShapeDtypeStruct((B, C_OUT), f32),
        ],
    )(gates, W2, b_exp)

    Wc3 = W_c.reshape(B, KS * KS * C_OUT, C_IN)
    bc3 = b_c.reshape(B, C_OUT, 1)
    FLAT = (H + 2) * (W + 2)
    PADL = (W + 2) + 1
    xp = jnp.pad(x, ((0, 0), (0, 0), (1, 1), (1, 1)))
    xpf = jnp.pad(xp.reshape(B, C_IN, FLAT), ((0, 0), (0, 0), (PADL, PADL)))
    y_flat = pl.pallas_call(
        _make_conv_body(C_IN, C_OUT, H, W, KS),
        grid=(B,),
        in_specs=[
            pl.BlockSpec((1, C_IN, FLAT + 2 * PADL), lambda b: (b, 0, 0)),
            pl.BlockSpec((1, KS * KS * C_OUT, C_IN), lambda b: (b, 0, 0)),
            pl.BlockSpec((1, C_OUT, 1), lambda b: (b, 0, 0)),
        ],
        out_specs=pl.BlockSpec((1, C_OUT, FLAT), lambda b: (b, 0, 0)),
        out_shape=jax.ShapeDtypeStruct((B, C_OUT, FLAT), f32),
    )(xpf, Wc3, bc3)
    y = y_flat.reshape(B, C_OUT, H + 2, W + 2)[:, :, 1:H + 1, 1:W + 1]

    return (y, loss_arr[0, 0])


# speedup vs baseline: 5.6488x; 1.7939x over previous
import jax
import jax.numpy as jnp
from jax.experimental import pallas as pl
from jax.experimental.pallas import tpu as pltpu


def _gating_body(x_ref, wg_ref, gates_ref, loss_ref, gx_ref):
    i = pl.program_id(0)
    n = pl.num_programs(0)
    gb = x_ref.shape[0]
    base = pl.multiple_of(i * gb, 8)
    gx_ref[pl.ds(base, gb), :] = jnp.mean(x_ref[...], axis=2)

    @pl.when(i == n - 1)
    def _():
        B = gx_ref.shape[0]
        E = wg_ref.shape[1]
        logits = jnp.dot(gx_ref[...], wg_ref[...],
                         preferred_element_type=jnp.float32)
        z = jnp.exp(logits - jnp.max(logits, axis=1, keepdims=True))
        sm = z / jnp.sum(z, axis=1, keepdims=True)
        iota = jax.lax.broadcasted_iota(jnp.int32, (B, E), 1)
        m1 = jnp.max(sm, axis=1, keepdims=True)
        i1 = jnp.min(jnp.where(sm == m1, iota, E), axis=1, keepdims=True)
        masked = jnp.where(iota == i1, -1.0, sm)
        m2 = jnp.max(masked, axis=1, keepdims=True)
        i2 = jnp.min(jnp.where(masked == m2, iota, E), axis=1, keepdims=True)
        denom = m1 + m2 + 1e-6
        gates = (jnp.where(iota == i1, m1 / denom, 0.0)
                 + jnp.where(iota == i2, m2 / denom, 0.0))
        gates_ref[...] = gates

        def cv_sq(v):
            mu = jnp.mean(v, keepdims=True)
            var = jnp.sum((v - mu) ** 2, keepdims=True) / (E - 1)
            return var / (mu ** 2 + 1e-10)

        imp = jnp.sum(gates, axis=0, keepdims=True)
        load = jnp.sum((gates > 0.0).astype(jnp.float32), axis=0,
                       keepdims=True)
        loss_ref[...] = (cv_sq(imp) + cv_sq(load)) * 0.01


def _combine_body(g_ref, w2_ref, be_ref, wc_ref, bc_ref):
    j = pl.program_id(0)
    wc_ref[...] = jnp.dot(g_ref[...], w2_ref[...],
                          preferred_element_type=jnp.float32)

    @pl.when(j == 0)
    def _():
        bc_ref[...] = jnp.dot(g_ref[...], be_ref[...],
                              preferred_element_type=jnp.float32)


def _make_conv_body(C_IN, C_OUT, H, W, KS):
    HW = H * W

    def conv_body(x_ref, wc_ref, bc_ref, out_ref):
        col = jax.lax.broadcasted_iota(jnp.int32, (1, HW), 1) % W
        mask_l = (col != 0).astype(jnp.float32)
        mask_r = (col != W - 1).astype(jnp.float32)

        wc = wc_ref[0]
        acc = None
        for t in range(KS * KS):
            kh, kw = t // KS, t % KS
            d = W * (kh - 1) + (kw - 1)
            wt = wc[t * C_OUT:(t + 1) * C_OUT, :]
            p = jnp.dot(wt, x_ref[0], preferred_element_type=jnp.float32)
            if d > 0:
                p = jnp.concatenate(
                    [p[:, d:], jnp.zeros((C_OUT, d), jnp.float32)], axis=1)
            elif d < 0:
                p = jnp.concatenate(
                    [jnp.zeros((C_OUT, -d), jnp.float32), p[:, :HW + d]],
                    axis=1)
            if kw == 0:
                p = p * mask_l
            elif kw == KS - 1:
                p = p * mask_r
            acc = p if acc is None else acc + p
        out_ref[0] = acc + bc_ref[0]

    return conv_body


def kernel(x, w_gate, W_exp, b_exp):
    B, C_IN, H, W = x.shape
    E = w_gate.shape[1]
    C_OUT = W_exp.shape[1]
    KS = W_exp.shape[3]
    HW = H * W
    f32 = jnp.float32

    GB = 8
    x3 = x.reshape(B, C_IN, HW)
    gates, loss_arr = pl.pallas_call(
        _gating_body,
        grid=(B // GB,),
        in_specs=[
            pl.BlockSpec((GB, C_IN, HW), lambda i: (i, 0, 0)),
            pl.BlockSpec((C_IN, E), lambda i: (0, 0)),
        ],
        out_specs=[
            pl.BlockSpec((B, E), lambda i: (0, 0)),
            pl.BlockSpec((1, 1), lambda i: (0, 0)),
        ],
        out_shape=[
            jax.ShapeDtypeStruct((B, E), f32),
            jax.ShapeDtypeStruct((1, 1), f32),
        ],
        scratch_shapes=[pltpu.VMEM((B, C_IN), f32)],
    )(x3, w_gate)

    W2 = W_exp.transpose(0, 3, 4, 1, 2).reshape(E, KS * KS * C_OUT * C_IN)
    NCH = 8
    CHUNK = W2.shape[1] // NCH
    W_c, b_c = pl.pallas_call(
        _combine_body,
        grid=(NCH,),
        in_specs=[
            pl.BlockSpec((B, E), lambda j: (0, 0)),
            pl.BlockSpec((E, CHUNK), lambda j: (0, j)),
            pl.BlockSpec((E, C_OUT), lambda j: (0, 0)),
        ],
        out_specs=[
            pl.BlockSpec((B, CHUNK), lambda j: (0, j)),
            pl.BlockSpec((B, C_OUT), lambda j: (0, 0)),
        ],
        out_shape=[
            jax.ShapeDtypeStruct((B, KS * KS * C_OUT * C_IN), f32),
            jax.ShapeDtypeStruct((B, C_OUT), f32),
        ],
    )(gates, W2, b_exp)

    Wc3 = W_c.reshape(B, KS * KS * C_OUT, C_IN)
    bc3 = b_c.reshape(B, C_OUT, 1)
    y_flat = pl.pallas_call(
        _make_conv_body(C_IN, C_OUT, H, W, KS),
        grid=(B,),
        in_specs=[
            pl.BlockSpec((1, C_IN, HW), lambda b: (b, 0, 0)),
            pl.BlockSpec((1, KS * KS * C_OUT, C_IN), lambda b: (b, 0, 0)),
            pl.BlockSpec((1, C_OUT, 1), lambda b: (b, 0, 0)),
        ],
        out_specs=pl.BlockSpec((1, C_OUT, HW), lambda b: (b, 0, 0)),
        out_shape=jax.ShapeDtypeStruct((B, C_OUT, HW), f32),
    )(x3, Wc3, bc3)
    y = y_flat.reshape(B, C_OUT, H, W)

    return (y, loss_arr[0, 0])


# speedup vs baseline: 5.7350x; 1.0152x over previous
import jax
import jax.numpy as jnp
from jax.experimental import pallas as pl
from jax.experimental.pallas import tpu as pltpu


def _gating_body(x_ref, wg_ref, gates_ref, loss_ref, gx_ref):
    i = pl.program_id(0)
    n = pl.num_programs(0)
    gb = x_ref.shape[0]
    base = pl.multiple_of(i * gb, 8)
    gx_ref[pl.ds(base, gb), :] = jnp.mean(x_ref[...], axis=2)

    @pl.when(i == n - 1)
    def _():
        B = gx_ref.shape[0]
        E = wg_ref.shape[1]
        logits = jnp.dot(gx_ref[...], wg_ref[...],
                         preferred_element_type=jnp.float32)
        z = jnp.exp(logits - jnp.max(logits, axis=1, keepdims=True))
        sm = z / jnp.sum(z, axis=1, keepdims=True)
        iota = jax.lax.broadcasted_iota(jnp.int32, (B, E), 1)
        m1 = jnp.max(sm, axis=1, keepdims=True)
        i1 = jnp.min(jnp.where(sm == m1, iota, E), axis=1, keepdims=True)
        masked = jnp.where(iota == i1, -1.0, sm)
        m2 = jnp.max(masked, axis=1, keepdims=True)
        i2 = jnp.min(jnp.where(masked == m2, iota, E), axis=1, keepdims=True)
        denom = m1 + m2 + 1e-6
        gates = (jnp.where(iota == i1, m1 / denom, 0.0)
                 + jnp.where(iota == i2, m2 / denom, 0.0))
        gates_ref[...] = gates

        def cv_sq(v):
            mu = jnp.mean(v, keepdims=True)
            var = jnp.sum((v - mu) ** 2, keepdims=True) / (E - 1)
            return var / (mu ** 2 + 1e-10)

        imp = jnp.sum(gates, axis=0, keepdims=True)
        load = jnp.sum((gates > 0.0).astype(jnp.float32), axis=0,
                       keepdims=True)
        loss_ref[...] = (cv_sq(imp) + cv_sq(load)) * 0.01


def _combine_body(g_ref, w2_ref, be_ref, wc_ref, bc_ref):
    j = pl.program_id(0)
    wc_ref[...] = jnp.dot(g_ref[...], w2_ref[...],
                          preferred_element_type=jnp.float32)

    @pl.when(j == 0)
    def _():
        bc_ref[...] = jnp.dot(g_ref[...], be_ref[...],
                              preferred_element_type=jnp.float32)


def _make_conv_body(C_IN, C_OUT, H, W, KS):
    HW = H * W

    def conv_body(x_ref, wc_ref, bc_ref, out_ref):
        col = jax.lax.broadcasted_iota(jnp.int32, (1, HW), 1) % W
        mask_l = (col != 0).astype(jnp.float32)
        mask_r = (col != W - 1).astype(jnp.float32)

        p_all = jnp.dot(wc_ref[0], x_ref[0],
                        preferred_element_type=jnp.float32)
        acc = None
        for t in range(KS * KS):
            kh, kw = t // KS, t % KS
            d = W * (kh - 1) + (kw - 1)
            p = p_all[t * C_OUT:(t + 1) * C_OUT, :]
            if d > 0:
                p = jnp.concatenate(
                    [p[:, d:], jnp.zeros((C_OUT, d), jnp.float32)], axis=1)
            elif d < 0:
                p = jnp.concatenate(
                    [jnp.zeros((C_OUT, -d), jnp.float32), p[:, :HW + d]],
                    axis=1)
            if kw == 0:
                p = p * mask_l
            elif kw == KS - 1:
                p = p * mask_r
            acc = p if acc is None else acc + p
        out_ref[0] = acc + bc_ref[0]

    return conv_body


def kernel(x, w_gate, W_exp, b_exp):
    B, C_IN, H, W = x.shape
    E = w_gate.shape[1]
    C_OUT = W_exp.shape[1]
    KS = W_exp.shape[3]
    HW = H * W
    f32 = jnp.float32

    GB = 8
    x3 = x.reshape(B, C_IN, HW)
    gates, loss_arr = pl.pallas_call(
        _gating_body,
        grid=(B // GB,),
        in_specs=[
            pl.BlockSpec((GB, C_IN, HW), lambda i: (i, 0, 0)),
            pl.BlockSpec((C_IN, E), lambda i: (0, 0)),
        ],
        out_specs=[
            pl.BlockSpec((B, E), lambda i: (0, 0)),
            pl.BlockSpec((1, 1), lambda i: (0, 0)),
        ],
        out_shape=[
            jax.ShapeDtypeStruct((B, E), f32),
            jax.ShapeDtypeStruct((1, 1), f32),
        ],
        scratch_shapes=[pltpu.VMEM((B, C_IN), f32)],
    )(x3, w_gate)

    W2 = W_exp.transpose(0, 3, 4, 1, 2).reshape(E, KS * KS * C_OUT * C_IN)
    NCH = 8
    CHUNK = W2.shape[1] // NCH
    W_c, b_c = pl.pallas_call(
        _combine_body,
        grid=(NCH,),
        in_specs=[
            pl.BlockSpec((B, E), lambda j: (0, 0)),
            pl.BlockSpec((E, CHUNK), lambda j: (0, j)),
            pl.BlockSpec((E, C_OUT), lambda j: (0, 0)),
        ],
        out_specs=[
            pl.BlockSpec((B, CHUNK), lambda j: (0, j)),
            pl.BlockSpec((B, C_OUT), lambda j: (0, 0)),
        ],
        out_shape=[
            jax.ShapeDtypeStruct((B, KS * KS * C_OUT * C_IN), f32),
            jax.ShapeDtypeStruct((B, C_OUT), f32),
        ],
    )(gates, W2, b_exp)

    Wc3 = W_c.reshape(B, KS * KS * C_OUT, C_IN)
    bc3 = b_c.reshape(B, C_OUT, 1)
    y_flat = pl.pallas_call(
        _make_conv_body(C_IN, C_OUT, H, W, KS),
        grid=(B,),
        in_specs=[
            pl.BlockSpec((1, C_IN, HW), lambda b: (b, 0, 0)),
            pl.BlockSpec((1, KS * KS * C_OUT, C_IN), lambda b: (b, 0, 0)),
            pl.BlockSpec((1, C_OUT, 1), lambda b: (b, 0, 0)),
        ],
        out_specs=pl.BlockSpec((1, C_OUT, HW), lambda b: (b, 0, 0)),
        out_shape=jax.ShapeDtypeStruct((B, C_OUT, HW), f32),
    )(x3, Wc3, bc3)
    y = y_flat.reshape(B, C_OUT, H, W)

    return (y, loss_arr[0, 0])
